# Initial kernel scaffold; baseline (speedup 1.0000x reference)
import numpy as np
import jax
import jax.numpy as jnp
from jax.experimental import pallas as pl

_MIN_GAP = 4
_MAX_GAP = 48
_REPEATED_SAMPLING = 4


def _temporal_indices(t: int):
    max_gap = min(_MAX_GAP, t - 1)
    gap = np.linspace(_MIN_GAP, max_gap, _REPEATED_SAMPLING).astype(np.int32)
    return [0] + [int(g) for g in gap]


def _copy_body(x_ref, o_ref):
    o_ref[...] = x_ref[...]


def kernel(x):
    b, c, t, h, w = x.shape
    idx = _temporal_indices(t)
    k = len(idx)
    bc = b * c
    hw = h * w
    xv = x.reshape(bc, t, hw)

    def in_map(i, j):
        src = sum(v * (j == kk) for kk, v in enumerate(idx))
        return (i, src, 0)

    out = pl.pallas_call(
        _copy_body,
        grid=(bc, k),
        in_specs=[pl.BlockSpec((1, 1, hw), in_map)],
        out_specs=pl.BlockSpec((1, 1, hw), lambda i, j: (i, j, 0)),
        out_shape=jax.ShapeDtypeStruct((bc, k, hw), x.dtype),
    )(xv)
    return out.reshape(b, c, k, h, w)


# speedup vs baseline: 1.5327x; 1.5327x over previous
import numpy as np
import jax
import jax.numpy as jnp
from jax.experimental import pallas as pl

_MIN_GAP = 4
_MAX_GAP = 48
_REPEATED_SAMPLING = 4


def _temporal_indices(t: int):
    max_gap = min(_MAX_GAP, t - 1)
    gap = np.linspace(_MIN_GAP, max_gap, _REPEATED_SAMPLING).astype(np.int32)
    return [0] + [int(g) for g in gap]


def _copy_body(x_ref, o_ref):
    o_ref[...] = x_ref[...]


def kernel(x):
    b, c, t, h, w = x.shape
    idx = _temporal_indices(t)
    k = len(idx)
    bc = b * c
    xv = x.reshape(bc, t, h, w)

    def in_map(i, j):
        src = sum(v * (j == kk) for kk, v in enumerate(idx))
        return (i, src, 0, 0)

    out = pl.pallas_call(
        _copy_body,
        grid=(bc, k),
        in_specs=[pl.BlockSpec((1, 1, h, w), in_map)],
        out_specs=pl.BlockSpec((1, 1, h, w), lambda i, j: (i, j, 0, 0)),
        out_shape=jax.ShapeDtypeStruct((bc, k, h, w), x.dtype),
    )(xv)
    return out.reshape(b, c, k, h, w)
